# Initial kernel scaffold; baseline (speedup 1.0000x reference)
import jax
import jax.numpy as jnp
from jax.experimental import pallas as pl


def kernel(indices, attention_weights, attention_accumulator, access_timestamps, current_time):
    raise NotImplementedError("write your pallas kernel here")


# speedup vs baseline: 2.7248x; 2.7248x over previous
import jax
import jax.numpy as jnp
from jax import lax
from jax.experimental import pallas as pl
from jax.experimental.pallas import tpu as pltpu
from jax.experimental.pallas import tpu_sc as plsc

_CACHE = 1_000_000
_B = 16_384
_NS = 16
_NCHUNK = 8
_LANE = 128
_STAGE = 62_496
_STAGE_LAST = _CACHE - (_NS - 1) * _STAGE


def _chunk_copy(src, dst, s, via):
    @pl.when(s < _NS - 1)
    def _():
        off = pl.multiple_of(s * _STAGE, 8)
        pltpu.sync_copy(src.at[pl.ds(off, _STAGE)], via.at[pl.ds(0, _STAGE)])
        pltpu.sync_copy(via.at[pl.ds(0, _STAGE)], dst.at[pl.ds(off, _STAGE)])

    @pl.when(s == _NS - 1)
    def _():
        off = (_NS - 1) * _STAGE
        pltpu.sync_copy(src.at[pl.ds(off, _STAGE_LAST)], via)
        pltpu.sync_copy(via, dst.at[pl.ds(off, _STAGE_LAST)])


def _sc_body(idx_hbm, w_hbm, acc_hbm, ts_hbm, ct_hbm,
             acc_out, ts_out, sh, stage_v, idx_v, w_v, ct_v):
    c = lax.axis_index("c")
    s = lax.axis_index("s")

    @pl.when(c == 0)
    def _():
        _chunk_copy(acc_hbm, sh, s, stage_v)

    @pl.when(c == 1)
    def _():
        _chunk_copy(ts_hbm, sh, s, stage_v)

    pltpu.sync_copy(idx_hbm.at[s], idx_v)

    @pl.when(c == 0)
    def _():
        pltpu.sync_copy(w_hbm.at[s], w_v)

    @pl.when(c == 1)
    def _():
        pltpu.sync_copy(ct_hbm, ct_v)

    plsc.subcore_barrier()

    for j in range(_NCHUNK):
        @pl.when(c == 0)
        def _():
            pltpu.sync_copy(w_v.at[j], sh.at[idx_v.at[j]], add=True)

        @pl.when(c == 1)
        def _():
            pltpu.sync_copy(ct_v, sh.at[idx_v.at[j]])

    plsc.subcore_barrier()

    @pl.when(c == 0)
    def _():
        _chunk_copy(sh, acc_out, s, stage_v)

    @pl.when(c == 1)
    def _():
        _chunk_copy(sh, ts_out, s, stage_v)


def _run(idx3, w3, acc, ts, ctv):
    f = pl.kernel(
        _sc_body,
        out_type=(jax.ShapeDtypeStruct((_CACHE,), jnp.float32),
                  jax.ShapeDtypeStruct((_CACHE,), jnp.float32)),
        mesh=plsc.VectorSubcoreMesh(core_axis_name="c", subcore_axis_name="s"),
        scratch_types=[
            pltpu.VMEM_SHARED((_CACHE,), jnp.float32),
            pltpu.VMEM((_STAGE_LAST,), jnp.float32),
            pltpu.VMEM((_NCHUNK, _LANE), jnp.int32),
            pltpu.VMEM((_NCHUNK, _LANE), jnp.float32),
            pltpu.VMEM((_LANE,), jnp.float32),
        ],
    )
    return f(idx3, w3, acc, ts, ctv)


def kernel(indices, attention_weights, attention_accumulator,
           access_timestamps, current_time):
    idx3 = indices.reshape(_NS, _NCHUNK, _LANE)
    w3 = attention_weights.reshape(_NS, _NCHUNK, _LANE)
    ctv = jnp.broadcast_to(current_time.astype(jnp.float32), (_LANE,))
    new_acc, new_ts = _run(idx3, w3, attention_accumulator,
                           access_timestamps, ctv)
    return new_acc, new_ts, current_time + 1
